# Initial kernel scaffold; baseline (speedup 1.0000x reference)
import jax
import jax.numpy as jnp
from jax.experimental import pallas as pl


def kernel(image, offsets):
    raise NotImplementedError("write your pallas kernel here")


# speedup vs baseline: 1.1836x; 1.1836x over previous
import functools

import jax
import jax.numpy as jnp
from jax import lax
from jax.experimental import pallas as pl
from jax.experimental.pallas import tpu as pltpu
from jax.experimental.pallas import tpu_sc as plsc

B, H, W, C = 8, 224, 224, 96
N = B * H * W
NW = 32
ROWS_PER_W = (B * H) // NW
LG = W // 16


def _floor(x):
    t = x.astype(jnp.int32)
    return jnp.where(x < t.astype(jnp.float32), t - 1, t)


def _sc_body(img_hbm, dx_hbm, dy_hbm, out_hbm,
             dxv, dyv, ia, ib, ic, id_, wav, wbv, wcv, wdv,
             bufa, bufb, bufc, bufd, outv, sem):
    wid = lax.axis_index("s") * 2 + lax.axis_index("c")
    b = lax.shift_right_logical(wid, 2)
    r0 = (wid & 3) * ROWS_PER_W
    bbase = b * (H * W)

    lanes = lax.iota(jnp.int32, 16)

    def chunk(g, carry):
        r = r0 + g
        p0 = bbase + r * W
        pltpu.sync_copy(dx_hbm.at[pl.ds(p0, W)], dxv)
        pltpu.sync_copy(dy_hbm.at[pl.ds(p0, W)], dyv)

        yb = r.astype(jnp.float32) + 1.0

        for i in range(LG):
            s = pl.ds(i * 16, 16)
            x = dxv[s] + (lanes + (i * 16 + 1)).astype(jnp.float32)
            y = dyv[s] + yb
            x = jnp.clip(x, -8.0, 232.0)
            y = jnp.clip(y, -8.0, 232.0)
            xf = _floor(x)
            yf = _floor(y)
            x0 = jnp.clip(xf, 0, W + 1)
            x1 = jnp.clip(xf + 1, 0, W + 1)
            y0 = jnp.clip(yf, 0, H + 1)
            y1 = jnp.clip(yf + 1, 0, H + 1)
            dxw = x1.astype(jnp.float32) - x
            dyw = y1.astype(jnp.float32) - y
            vx0 = (x0 >= 1) & (x0 <= W)
            vx1 = (x1 >= 1) & (x1 <= W)
            vy0 = (y0 >= 1) & (y0 <= H)
            vy1 = (y1 >= 1) & (y1 <= H)
            zero = jnp.zeros((16,), jnp.float32)
            wa = jnp.where(vx0 & vy0, dxw * dyw, zero)
            wb = jnp.where(vx0 & vy1, dxw * (1.0 - dyw), zero)
            wc = jnp.where(vx1 & vy0, (1.0 - dxw) * dyw, zero)
            wd = jnp.where(vx1 & vy1, (1.0 - dxw) * (1.0 - dyw), zero)
            xu0 = jnp.clip(x0 - 1, 0, W - 1)
            xu1 = jnp.clip(x1 - 1, 0, W - 1)
            yu0 = jnp.clip(y0 - 1, 0, H - 1) * W + bbase
            yu1 = jnp.clip(y1 - 1, 0, H - 1) * W + bbase
            j, col = divmod(i * 16, 112)
            cs = pl.ds(col, 16)
            ia[j, cs] = yu0 + xu0
            ib[j, cs] = yu1 + xu0
            ic[j, cs] = yu0 + xu1
            id_[j, cs] = yu1 + xu1
            wav[s] = wa
            wbv[s] = wb
            wcv[s] = wc
            wdv[s] = wd

        copies = []
        for idx, buf in ((ia, bufa), (ib, bufb), (ic, bufc), (id_, bufd)):
            for j in range(2):
                copies.append(pltpu.async_copy(
                    img_hbm.at[idx.at[j]],
                    buf.at[pl.ds(j * 112, 112), :], sem))
        for cp in copies:
            cp.wait()

        def combine(i, carry2):
            s16 = pl.ds(i * 16, 16)
            wa16 = wav[s16]
            wb16 = wbv[s16]
            wc16 = wcv[s16]
            wd16 = wdv[s16]
            k0 = i * 16
            for k2 in range(16):
                k = k0 + k2
                wa = wa16[k2]
                wb = wb16[k2]
                wc = wc16[k2]
                wd = wd16[k2]
                for jj in range(C // 16):
                    s = pl.ds(jj * 16, 16)
                    outv[k, s] = (wa * bufa[k, s] + wb * bufb[k, s]
                                  + wc * bufc[k, s] + wd * bufd[k, s])
            return carry2

        lax.fori_loop(0, LG, combine, 0)
        pltpu.sync_copy(outv, out_hbm.at[pl.ds(p0, W), :])
        return carry

    lax.fori_loop(0, ROWS_PER_W, chunk, 0)


@jax.jit
def _run(img_flat, dx, dy):
    kern = functools.partial(
        pl.kernel,
        mesh=plsc.VectorSubcoreMesh(core_axis_name="c", subcore_axis_name="s"),
        out_type=jax.ShapeDtypeStruct((N, C), jnp.float32),
        scratch_types=[
            pltpu.VMEM((W,), jnp.float32),
            pltpu.VMEM((W,), jnp.float32),
            pltpu.VMEM((2, 112), jnp.int32),
            pltpu.VMEM((2, 112), jnp.int32),
            pltpu.VMEM((2, 112), jnp.int32),
            pltpu.VMEM((2, 112), jnp.int32),
            pltpu.VMEM((W,), jnp.float32),
            pltpu.VMEM((W,), jnp.float32),
            pltpu.VMEM((W,), jnp.float32),
            pltpu.VMEM((W,), jnp.float32),
            pltpu.VMEM((W, C), jnp.float32),
            pltpu.VMEM((W, C), jnp.float32),
            pltpu.VMEM((W, C), jnp.float32),
            pltpu.VMEM((W, C), jnp.float32),
            pltpu.VMEM((W, C), jnp.float32),
            pltpu.SemaphoreType.DMA,
        ],
        compiler_params=pltpu.CompilerParams(use_tc_tiling_on_sc=False),
    )(_sc_body)
    return kern(img_flat, dx, dy)


def kernel(image, offsets):
    img_flat = image.reshape(N, C)
    dx = offsets[..., 0].reshape(N)
    dy = offsets[..., 1].reshape(N)
    out = _run(img_flat, dx, dy)
    return out.reshape(B, H, W, C)
